# baseline (speedup 1.0000x reference)
import functools

import jax
import jax.numpy as jnp
from jax import lax
from jax.experimental import pallas as pl
from jax.experimental.pallas import tpu as pltpu
from jax.experimental.pallas import tpu_sc as plsc

D = 32
DB = D // 8
NC, NS = 2, 16
NW = NC * NS
BATCH = 128
STEPS = 200
SB = STEPS // 8

_mesh = plsc.VectorSubcoreMesh(core_axis_name="c", subcore_axis_name="s")


@functools.partial(
    pl.kernel,
    out_type=jax.ShapeDtypeStruct((STEPS, DB, NW, 8, BATCH), jnp.float32),
    mesh=_mesh,
    compiler_params=pltpu.CompilerParams(
        use_tc_tiling_on_sc=False,
        needs_layout_passes=False,
        disable_bounds_checks=True,
    ),
    scratch_types=[
        pltpu.VMEM((SB, 8, BATCH), jnp.int32),
        pltpu.VMEM((2, BATCH, D), jnp.float32),
        pltpu.VMEM((2, DB, 8, BATCH), jnp.float32),
        pltpu.SemaphoreType.DMA,
        pltpu.SemaphoreType.DMA,
        pltpu.SemaphoreType.DMA,
        pltpu.SemaphoreType.DMA,
        pltpu.SemaphoreType.DMA,
    ],
)
def _emb_gather(idx_hbm, table_hbm, out_hbm, idx_v, rows_v, trows_v,
                isem, gsem0, gsem1, osem0, osem1):
    wid = lax.axis_index("s") * NC + lax.axis_index("c")
    for sb in range(SB):
        pltpu.async_copy(idx_hbm.at[sb, wid], idx_v.at[sb], isem)
    for sb in range(SB):
        pltpu.make_async_copy(idx_hbm.at[sb, wid], idx_v.at[sb], isem).wait()

    gsems = (gsem0, gsem1)
    osems = (osem0, osem1)
    iota = lax.iota(jnp.int32, 16)
    jidx = tuple(iota + j0 for j0 in range(0, BATCH, 16))

    def fire(s, b):
        row = idx_v.at[lax.div(s, 8), lax.rem(s, 8)]
        pltpu.async_copy(table_hbm.at[row], rows_v.at[b], gsems[b])

    def drain_gather(b):
        pltpu.make_async_copy(
            table_hbm.at[pl.ds(0, BATCH)], rows_v.at[b], gsems[b]
        ).wait()

    def transpose(b):
        def dbody(d, carry):
            db = lax.div(d, 8)
            di = lax.rem(d, 8)
            didx = jnp.full((16,), 0, jnp.int32) + d
            for k in range(8):
                v = plsc.load_gather(rows_v.at[b], [jidx[k], didx])
                trows_v[b, db, di, pl.ds(k * 16, 16)] = v
            return carry

        lax.fori_loop(0, D, dbody, 0)

    def start_out(s, b):
        for db in range(DB):
            pltpu.async_copy(
                trows_v.at[b, db], out_hbm.at[s, db, wid], osems[b]
            )

    def drain_out(b):
        for db in range(DB):
            pltpu.make_async_copy(
                trows_v.at[b, db], out_hbm.at[0, db, 0], osems[b]
            ).wait()

    def step(s, b, first):
        if not first:
            drain_out(b)
        drain_gather(b)
        transpose(b)
        if not isinstance(s, int) or s + 2 < STEPS:
            fire(s + 2, b)
        start_out(s, b)

    fire(0, 0)
    fire(1, 1)
    step(0, 0, True)
    step(1, 1, True)

    def pair(t, carry):
        step(2 * t, 0, False)
        step(2 * t + 1, 1, False)
        return carry

    lax.fori_loop(1, STEPS // 2 - 1, pair, 0)

    step(STEPS - 2, 0, False)
    step(STEPS - 1, 1, False)
    drain_out(0)
    drain_out(1)


def kernel(token_ids, embeddings):
    b, s = token_ids.shape
    idx = token_ids.T.reshape(SB, 8, NW, BATCH).transpose(0, 2, 1, 3)
    out5 = _emb_gather(idx, embeddings)
    return out5.transpose(2, 4, 0, 1, 3).reshape(b, s, D)
